# baseline (speedup 1.0000x reference)
import functools

import jax
import jax.numpy as jnp
from jax import lax
from jax.experimental import pallas as pl
from jax.experimental.pallas import tpu as pltpu
from jax.experimental.pallas import tpu_sc as plsc

NUM_EXPERTS = 8
TOP_K = 2
D_MODEL = 1024
D_FF = 512
T_TOK = 2048
NSLOT = T_TOK * TOP_K
BBLK = 256
CAP = NSLOT + NUM_EXPERTS * BBLK
NB = CAP // BBLK
CHUNK = 256
NW = 32
TKW = T_TOK // NW
SLW = NSLOT // NW


def _dot(a, b):
    return jax.lax.dot_general(
        a, b, (((1,), (0,)), ((), ())), preferred_element_type=jnp.float32
    )


def _router_kernel(x_ref, gate_ref, dest_ref, tw_ref, bmeta_ref):
    x = x_ref[...]

    logits = _dot(x, gate_ref[...])
    m = jnp.max(logits, axis=-1, keepdims=True)
    ee = jnp.exp(logits - m)
    probs = ee / jnp.sum(ee, axis=-1, keepdims=True)

    iota = lax.broadcasted_iota(jnp.int32, probs.shape, 1)
    w1 = jnp.max(probs, axis=-1, keepdims=True)
    is1 = probs == w1
    i1 = jnp.min(jnp.where(is1, iota, NUM_EXPERTS), axis=-1, keepdims=True)
    mask1 = iota == i1
    probs2 = jnp.where(mask1, -jnp.inf, probs)
    w2 = jnp.max(probs2, axis=-1, keepdims=True)
    is2 = probs2 == w2
    i2 = jnp.min(jnp.where(is2, iota, NUM_EXPERTS), axis=-1, keepdims=True)
    mask2 = iota == i2
    denom = w1 + w2
    tw_ref[...] = jnp.concatenate([w1 / denom, w2 / denom], axis=1)

    cnt = (mask1 | mask2).astype(jnp.float32)
    r = lax.broadcasted_iota(jnp.int32, (CHUNK, CHUNK), 0)
    c = lax.broadcasted_iota(jnp.int32, (CHUNK, CHUNK), 1)
    ltri = (c < r).astype(jnp.bfloat16)
    run = jnp.zeros((1, NUM_EXPERTS), jnp.float32)
    parts = []
    for ch in range(T_TOK // CHUNK):
        cnt_c = cnt[ch * CHUNK:(ch + 1) * CHUNK]
        cc = _dot(ltri, cnt_c.astype(jnp.bfloat16))
        parts.append(cc + run)
        run = run + jnp.sum(cnt_c, axis=0, keepdims=True)
    colcum = jnp.concatenate(parts, axis=0)
    counts = run

    pcount = jnp.floor((counts + (BBLK - 1)) * (1.0 / BBLK)) * BBLK
    ur = lax.broadcasted_iota(jnp.int32, (NUM_EXPERTS, NUM_EXPERTS), 0)
    uc = lax.broadcasted_iota(jnp.int32, (NUM_EXPERTS, NUM_EXPERTS), 1)
    utri = (ur < uc).astype(jnp.bfloat16)
    base = _dot(pcount.astype(jnp.bfloat16), utri)

    dval = base + colcum
    d0 = jnp.sum(jnp.where(mask1, dval, 0.0), axis=-1, keepdims=True)
    d1 = jnp.sum(jnp.where(mask2, dval, 0.0), axis=-1, keepdims=True)
    dest_ref[...] = jnp.concatenate([d0, d1], axis=1).astype(jnp.int32)

    eidx = lax.broadcasted_iota(jnp.int32, (1, NUM_EXPERTS), 1).astype(jnp.float32)
    bes = []
    for b in range(NB):
        lo = float(b * BBLK)
        ind = (lo >= base) & (lo < base + pcount)
        bes.append(jnp.sum(jnp.where(ind, eidx, 0.0), axis=-1, keepdims=True))
    nb = jnp.sum(pcount, axis=-1, keepdims=True) * (1.0 / BBLK)
    pad = jnp.zeros((1, 7), jnp.float32)
    bmeta_ref[...] = jnp.concatenate(bes + [nb, pad], axis=1).astype(jnp.int32)


def _k1_call(hidden_states, gate_w):
    return pl.pallas_call(
        _router_kernel,
        in_specs=[
            pl.BlockSpec((T_TOK, D_MODEL), lambda: (0, 0)),
            pl.BlockSpec((D_MODEL, NUM_EXPERTS), lambda: (0, 0)),
        ],
        out_specs=[
            pl.BlockSpec((T_TOK, 2), lambda: (0, 0)),
            pl.BlockSpec((T_TOK, 2), lambda: (0, 0)),
            pl.BlockSpec((1, 32), lambda: (0, 0)),
        ],
        out_shape=[
            jax.ShapeDtypeStruct((T_TOK, 2), jnp.int32),
            jax.ShapeDtypeStruct((T_TOK, 2), jnp.float32),
            jax.ShapeDtypeStruct((1, 32), jnp.int32),
        ],
    )(hidden_states, gate_w)


def _k2_scatter(x_bf3, dest0, dest1):
    mesh = plsc.VectorSubcoreMesh(core_axis_name="c", subcore_axis_name="s")

    @functools.partial(
        pl.kernel, mesh=mesh,
        out_type=jax.ShapeDtypeStruct((CAP, 8, 128), jnp.float32),
        scratch_types=[
            pltpu.VMEM((2, TKW), jnp.int32),
            pltpu.VMEM((TKW, 8, 128), jnp.float32),
            pltpu.SemaphoreType.DMA,
        ],
    )
    def k(x_hbm, d0_hbm, d1_hbm, xp_hbm, idx_v, rows_v, sem):
        wid = lax.axis_index("s") * 2 + lax.axis_index("c")
        base = wid * TKW
        pltpu.sync_copy(d0_hbm.at[pl.ds(base, TKW)], idx_v.at[0])
        pltpu.sync_copy(d1_hbm.at[pl.ds(base, TKW)], idx_v.at[1])
        pltpu.sync_copy(x_hbm.at[pl.ds(base, TKW)], rows_v)
        pltpu.async_copy(rows_v, xp_hbm.at[idx_v.at[0]], sem).wait()
        pltpu.async_copy(rows_v, xp_hbm.at[idx_v.at[1]], sem).wait()

    return k(x_bf3, dest0, dest1)


def _ffn_kernel(bmeta_ref, x_ref, w1_ref, b1_ref, w2_ref, b2_ref, out_ref):
    i = pl.program_id(0)
    nb = bmeta_ref[0, NB]

    @pl.when(i < nb)
    def _():
        ex = bmeta_ref[0, i]
        x = x_ref[...]
        h = _dot(x, w1_ref[ex]) + b1_ref[ex]
        h = jax.nn.gelu(h)
        y = _dot(h, w2_ref[ex]) + b2_ref[ex]
        out_ref[...] = y


def _k3_call(bmeta, x_perm2, W1, b1, W2, b2):
    full = lambda *shape: pl.BlockSpec(shape, lambda i: (0,) * len(shape))
    return pl.pallas_call(
        _ffn_kernel,
        grid=(NB,),
        in_specs=[
            pl.BlockSpec(memory_space=pltpu.SMEM),
            pl.BlockSpec((BBLK, D_MODEL), lambda i: (i, 0)),
            full(NUM_EXPERTS, D_MODEL, D_FF),
            full(NUM_EXPERTS, 1, D_FF),
            full(NUM_EXPERTS, D_FF, D_MODEL),
            full(NUM_EXPERTS, 1, D_MODEL),
        ],
        out_specs=pl.BlockSpec((BBLK, D_MODEL), lambda i: (i, 0)),
        out_shape=jax.ShapeDtypeStruct((CAP, D_MODEL), jnp.float32),
    )(bmeta, x_perm2, W1, b1.reshape(NUM_EXPERTS, 1, D_FF), W2,
      b2.reshape(NUM_EXPERTS, 1, D_MODEL))


def _k4_gather(y_perm3, destflat):
    mesh = plsc.VectorSubcoreMesh(core_axis_name="c", subcore_axis_name="s")

    half = SLW // 2

    @functools.partial(
        pl.kernel, mesh=mesh,
        out_type=jax.ShapeDtypeStruct((NSLOT, 8, 128), jnp.float32),
        scratch_types=[
            pltpu.VMEM((half,), jnp.int32),
            pltpu.VMEM((half,), jnp.int32),
            pltpu.VMEM((half, 8, 128), jnp.float32),
            pltpu.SemaphoreType.DMA,
        ],
    )
    def k(yp_hbm, df_hbm, ys_hbm, i0_v, i1_v, rows_v, sem):
        wid = lax.axis_index("s") * 2 + lax.axis_index("c")
        base = wid * SLW
        pltpu.sync_copy(df_hbm.at[pl.ds(base, half)], i0_v)
        pltpu.sync_copy(df_hbm.at[pl.ds(base + half, half)], i1_v)
        pltpu.async_copy(yp_hbm.at[i0_v], rows_v, sem).wait()
        pltpu.sync_copy(rows_v, ys_hbm.at[pl.ds(base, half)])
        pltpu.async_copy(yp_hbm.at[i1_v], rows_v, sem).wait()
        pltpu.sync_copy(rows_v, ys_hbm.at[pl.ds(base + half, half)])

    return k(y_perm3, destflat)


K5_TB = 512


def _combine_kernel(x_ref, ys_ref, tw_ref, sw1_ref, sb1_ref, sw2_ref,
                    sb2_ref, sgw_ref, sgb_ref, out_ref):
    x = x_ref[...]
    hs = _dot(x, sw1_ref[...]) + sb1_ref[...]
    hs = jax.nn.gelu(hs)
    ysh = _dot(hs, sw2_ref[...]) + sb2_ref[...]
    glog = _dot(x, sgw_ref[...]) + sgb_ref[...]
    g = jax.nn.sigmoid(glog)
    w0 = tw_ref[:, 0:1]
    w1 = tw_ref[:, 1:2]
    out_ref[...] = w0 * ys_ref[0] + w1 * ys_ref[1] + g * ysh


def _k5_call(hidden_states, y_slot, tw, shared_W1, sb1_2d, shared_W2,
             sb2_2d, sgate_w, sgb_2d):
    full = lambda *shape: pl.BlockSpec(shape, lambda i: (0,) * len(shape))
    return pl.pallas_call(
        _combine_kernel,
        grid=(T_TOK // K5_TB,),
        in_specs=[
            pl.BlockSpec((K5_TB, D_MODEL), lambda i: (i, 0)),
            pl.BlockSpec((2, K5_TB, D_MODEL), lambda i: (0, i, 0)),
            pl.BlockSpec((K5_TB, 2), lambda i: (i, 0)),
            full(D_MODEL, D_FF),
            full(1, D_FF),
            full(D_FF, D_MODEL),
            full(1, D_MODEL),
            full(D_MODEL, 1),
            full(1, 1),
        ],
        out_specs=pl.BlockSpec((K5_TB, D_MODEL), lambda i: (i, 0)),
        out_shape=jax.ShapeDtypeStruct((T_TOK, D_MODEL), jnp.float32),
    )(hidden_states, y_slot, tw, shared_W1, sb1_2d, shared_W2, sb2_2d,
      sgate_w, sgb_2d)


@jax.jit
def kernel(hidden_states, gate_w, W1, b1, W2, b2, shared_W1, shared_b1,
           shared_W2, shared_b2, sgate_w, sgate_b):
    dest, tw, bmeta = _k1_call(hidden_states, gate_w)

    x3 = hidden_states.reshape(T_TOK, 8, 128)
    dest0 = dest[:, 0]
    dest1 = dest[:, 1]
    x_perm3 = _k2_scatter(x3, dest0, dest1)

    y_perm2 = _k3_call(bmeta, x_perm3.reshape(CAP, D_MODEL), W1, b1, W2, b2)

    destflat = jnp.concatenate([dest0, dest1], axis=0)
    y_slot3 = _k4_gather(y_perm2.reshape(CAP, 8, 128), destflat)

    y_slot = y_slot3.reshape(2, T_TOK, D_MODEL)
    sb1_2d = shared_b1.reshape(1, D_FF)
    sb2_2d = shared_b2.reshape(1, D_MODEL)
    sgb_2d = sgate_b.reshape(1, 1)
    return _k5_call(hidden_states, y_slot, tw, shared_W1, sb1_2d,
                    shared_W2, sb2_2d, sgate_w, sgb_2d)


# speedup vs baseline: 2.4199x; 2.4199x over previous
import jax
import jax.numpy as jnp
from jax.experimental import pallas as pl
from jax.experimental.pallas import tpu as pltpu

NUM_EXPERTS = 8
TOP_K = 2
D_MODEL = 1024
D_FF = 512
T_TOK = 2048


def _dot(a, b):
    return jax.lax.dot_general(
        a, b, (((1,), (0,)), ((), ())), preferred_element_type=jnp.float32
    )


def _moe_kernel(x_ref, gate_ref, w1_ref, b1_ref, w2_ref, b2_ref,
                sw1_ref, sb1_ref, sw2_ref, sb2_ref, sgw_ref, sgb_ref,
                out_ref, comb_ref):
    e = pl.program_id(0)
    x = x_ref[...]

    @pl.when(e == 0)
    def _():
        logits = _dot(x, gate_ref[...])
        m = jnp.max(logits, axis=-1, keepdims=True)
        ex = jnp.exp(logits - m)
        probs = ex / jnp.sum(ex, axis=-1, keepdims=True)

        iota = jax.lax.broadcasted_iota(jnp.int32, probs.shape, 1)
        w1 = jnp.max(probs, axis=-1, keepdims=True)
        is1 = probs == w1
        i1 = jnp.min(jnp.where(is1, iota, NUM_EXPERTS), axis=-1, keepdims=True)
        mask1 = iota == i1
        probs2 = jnp.where(mask1, -jnp.inf, probs)
        w2 = jnp.max(probs2, axis=-1, keepdims=True)
        is2 = probs2 == w2
        i2 = jnp.min(jnp.where(is2, iota, NUM_EXPERTS), axis=-1, keepdims=True)
        mask2 = iota == i2
        comb_ref[...] = jnp.where(mask1 | mask2, probs, 0.0) / (w1 + w2)

        hs = _dot(x, sw1_ref[...]) + sb1_ref[...]
        hs = jax.nn.gelu(hs)
        ys = _dot(hs, sw2_ref[...]) + sb2_ref[...]
        g = jax.nn.sigmoid(_dot(x, sgw_ref[...]) + sgb_ref[...])
        out_ref[...] = g * ys

    h = _dot(x, w1_ref[0]) + b1_ref[0]
    h = jax.nn.gelu(h)
    y = _dot(h, w2_ref[0]) + b2_ref[0]
    iota = jax.lax.broadcasted_iota(jnp.int32, (T_TOK, NUM_EXPERTS), 1)
    sel = jnp.sum(jnp.where(iota == e, comb_ref[...], 0.0), axis=-1,
                  keepdims=True)
    out_ref[...] = out_ref[...] + sel * y


@jax.jit
def kernel(hidden_states, gate_w, W1, b1, W2, b2, shared_W1, shared_b1,
           shared_W2, shared_b2, sgate_w, sgate_b):
    T, D = hidden_states.shape

    sb1_2d = shared_b1.reshape(1, D_FF)
    sb2_2d = shared_b2.reshape(1, D_MODEL)
    sgb_2d = sgate_b.reshape(1, 1)

    full = lambda *shape: pl.BlockSpec(shape, lambda e: (0,) * len(shape))
    out = pl.pallas_call(
        _moe_kernel,
        grid=(NUM_EXPERTS,),
        in_specs=[
            full(T, D),
            full(D, NUM_EXPERTS),
            pl.BlockSpec((1, D, D_FF), lambda e: (e, 0, 0)),
            pl.BlockSpec((1, 1, D_FF), lambda e: (e, 0, 0)),
            pl.BlockSpec((1, D_FF, D), lambda e: (e, 0, 0)),
            pl.BlockSpec((1, 1, D), lambda e: (e, 0, 0)),
            full(D, D_FF),
            full(1, D_FF),
            full(D_FF, D),
            full(1, D),
            full(D, 1),
            full(1, 1),
        ],
        out_specs=pl.BlockSpec((T, D), lambda e: (0, 0)),
        out_shape=jax.ShapeDtypeStruct((T, D), jnp.float32),
        scratch_shapes=[pltpu.VMEM((T_TOK, NUM_EXPERTS), jnp.float32)],
    )(hidden_states, gate_w, W1, b1.reshape(NUM_EXPERTS, 1, D_FF), W2,
      b2.reshape(NUM_EXPERTS, 1, D_MODEL), shared_W1, sb1_2d,
      shared_W2, sb2_2d, sgate_w, sgb_2d)
    return out


# speedup vs baseline: 2.8106x; 1.1614x over previous
import jax
import jax.numpy as jnp
from jax.experimental import pallas as pl

NUM_EXPERTS = 8
TOP_K = 2
D_MODEL = 1024
D_FF = 512
TB = 512


def _dot(a, b):
    return jax.lax.dot_general(
        a, b, (((1,), (0,)), ((), ())), preferred_element_type=jnp.float32
    )


def _moe_block_kernel(x_ref, gate_ref, w1_ref, b1_ref, w2_ref, b2_ref,
                      sw1_ref, sb1_ref, sw2_ref, sb2_ref, sgw_ref, sgb_ref,
                      out_ref):
    x = x_ref[...]

    logits = _dot(x, gate_ref[...])
    m = jnp.max(logits, axis=-1, keepdims=True)
    e = jnp.exp(logits - m)
    probs = e / jnp.sum(e, axis=-1, keepdims=True)

    iota = jax.lax.broadcasted_iota(jnp.int32, probs.shape, 1)
    w1 = jnp.max(probs, axis=-1, keepdims=True)
    is_max = probs == w1
    i1 = jnp.min(jnp.where(is_max, iota, NUM_EXPERTS), axis=-1, keepdims=True)
    mask1 = iota == i1
    probs2 = jnp.where(mask1, -jnp.inf, probs)
    w2 = jnp.max(probs2, axis=-1, keepdims=True)
    is_max2 = probs2 == w2
    i2 = jnp.min(jnp.where(is_max2, iota, NUM_EXPERTS), axis=-1, keepdims=True)
    mask2 = iota == i2
    denom = w1 + w2
    combine = jnp.where(mask1 | mask2, probs, 0.0) / denom

    acc = jnp.zeros((TB, D_MODEL), jnp.float32)
    for ex in range(NUM_EXPERTS):
        h = _dot(x, w1_ref[ex]) + b1_ref[ex][None, :]
        h = jax.nn.gelu(h.astype(jnp.bfloat16))
        y = _dot(h, w2_ref[ex].astype(jnp.bfloat16)) + b2_ref[ex][None, :]
        acc = acc + combine[:, ex:ex + 1] * y

    hs = _dot(x, sw1_ref[...]) + sb1_ref[...]
    hs = jax.nn.gelu(hs.astype(jnp.bfloat16))
    ys = _dot(hs, sw2_ref[...].astype(jnp.bfloat16)) + sb2_ref[...]
    glog = _dot(x, sgw_ref[...]) + sgb_ref[...]
    g = jax.nn.sigmoid(glog)
    out_ref[...] = acc + g * ys


@jax.jit
def kernel(hidden_states, gate_w, W1, b1, W2, b2, shared_W1, shared_b1,
           shared_W2, shared_b2, sgate_w, sgate_b):
    T, D = hidden_states.shape
    num_blocks = T // TB

    sb1_2d = shared_b1.reshape(1, D_FF)
    sb2_2d = shared_b2.reshape(1, D_MODEL)
    sgb_2d = sgate_b.reshape(1, 1)

    full = lambda *shape: pl.BlockSpec(shape, lambda i: (0,) * len(shape))
    out = pl.pallas_call(
        _moe_block_kernel,
        grid=(num_blocks,),
        in_specs=[
            pl.BlockSpec((TB, D), lambda i: (i, 0)),
            full(D, NUM_EXPERTS),
            full(NUM_EXPERTS, D, D_FF),
            full(NUM_EXPERTS, D_FF),
            full(NUM_EXPERTS, D_FF, D),
            full(NUM_EXPERTS, D),
            full(D, D_FF),
            full(1, D_FF),
            full(D_FF, D),
            full(1, D),
            full(D, 1),
            full(1, 1),
        ],
        out_specs=pl.BlockSpec((TB, D), lambda i: (i, 0)),
        out_shape=jax.ShapeDtypeStruct((T, D), jnp.float32),
    )(hidden_states, gate_w, W1, b1, W2, b2, shared_W1, sb1_2d,
      shared_W2, sb2_2d, sgate_w, sgb_2d)
    return out
